# baseline (speedup 1.0000x reference)
import functools

import jax
import jax.numpy as jnp
from jax import lax
from jax.experimental import pallas as pl
from jax.experimental.pallas import tpu as pltpu
from jax.experimental.pallas import tpu_sc as plsc

_N = 100000
_E = 1600000
_BD = 8
_RF = 16
_NC = 2
_NS = 16
_NW = _NC * _NS
_EPT = _E // _NW
_C = 400
_NCHUNK = _EPT // _C
_PAIRS = (_NCHUNK - 1) // 2
_G = _C // 16
_SUB = 50
_NSUB = _C // _SUB
_IPT = _EPT // _SUB
_NSTAGE = 10
_RPT = _N // _NSTAGE
_SC = 400


def _compute_chunk(rs_v, rd_v, ps_v, pd_v, cs_v, cd_v, iota):
    def group_body(g, carry):
        rows = g * 16 + iota
        cols = [jnp.full((16,), j, jnp.int32) for j in range(16)]
        Rs = [plsc.load_gather(rs_v, [rows, cols[j // 4], cols[j % 4]])
              for j in range(16)]
        Rd = [plsc.load_gather(rd_v, [rows, cols[j // 4], cols[j % 4]])
              for j in range(16)]
        Ps = [plsc.load_gather(ps_v, [rows, cols[j]]) for j in range(8)]
        Pd = [plsc.load_gather(pd_v, [rows, cols[j]]) for j in range(8)]
        for b in range(2):
            r = []
            for a in range(4):
                t = Rs[a * 4] * Ps[b * 4]
                u = Rd[a * 4] * Pd[b * 4]
                for d in range(1, 4):
                    t = t + Rs[a * 4 + d] * Ps[b * 4 + d]
                    u = u + Rd[a * 4 + d] * Pd[b * 4 + d]
                r.append(t - u)
            for d in range(4):
                cs = Rs[d] * r[0]
                cd = Rd[d] * r[0]
                for a in range(1, 4):
                    cs = cs + Rs[a * 4 + d] * r[a]
                    cd = cd + Rd[a * 4 + d] * r[a]
                plsc.store_scatter(cs_v, [rows, cols[b * 4 + d]], cs)
                plsc.store_scatter(cd_v, [rows, cols[b * 4 + d]], -cd)
        return carry

    lax.fori_loop(0, _G, group_body, 0)


def _sheaf_step_body(p_hbm, srcr, dstr, rs_hbm, rd_hbm, zero_hbm, out_hbm,
                     acc,
                     igA, igB, isA, isB,
                     rsA, rdA, rsB, rdB,
                     psA, pdA, psB, pdB,
                     csA, cdA, csB, cdB,
                     s_igA, s_igB, s_isA, s_isB,
                     s_inA, s_inB, s_scA, s_scB):
    c = lax.axis_index("c")
    s = lax.axis_index("s")
    w = c * _NS + s

    r0 = s * _RPT

    @pl.when(s < _NSTAGE)
    def _():
        def zero_body(j, carry):
            rr = r0 + j * _SC
            pltpu.sync_copy(zero_hbm.at[pl.ds(rr, _SC)], csA)
            pltpu.sync_copy(csA, acc.at[pl.ds(rr, _SC)])
            return carry

        lax.fori_loop(0, _RPT // _SC, zero_body, 0)

    plsc.subcore_barrier()

    iota = lax.iota(jnp.int32, 16)
    e0 = w * _EPT
    ir0 = w * _IPT
    last = _NCHUNK - 1

    bufs = {
        0: (igA, isA, rsA, rdA, psA, pdA, csA, cdA, s_igA, s_isA, s_inA, s_scA),
        1: (igB, isB, rsB, rdB, psB, pdB, csB, cdB, s_igB, s_isB, s_inB, s_scB),
    }

    def issue_inputs(q, ph):
        ig, _, rs_v, rd_v, ps_v, pd_v, _, _, _, _, s_in, _ = bufs[ph]
        erow = e0 + q * _C
        pltpu.async_copy(rs_hbm.at[pl.ds(erow, _C)], rs_v, s_in)
        pltpu.async_copy(rd_hbm.at[pl.ds(erow, _C)], rd_v, s_in)
        for j in range(_NSUB):
            pltpu.async_copy(p_hbm.at[ig.at[j]],
                             ps_v.at[pl.ds(j * _SUB, _SUB)], s_in)
            pltpu.async_copy(p_hbm.at[ig.at[_NSUB + j]],
                             pd_v.at[pl.ds(j * _SUB, _SUB)], s_in)

    def wait_inputs(ph):
        ig, _, rs_v, rd_v, ps_v, pd_v, _, _, _, _, s_in, _ = bufs[ph]
        pltpu.make_async_copy(rs_hbm.at[pl.ds(0, _C)], rs_v, s_in).wait()
        pltpu.make_async_copy(rd_hbm.at[pl.ds(0, _C)], rd_v, s_in).wait()
        for j in range(_NSUB):
            pltpu.make_async_copy(p_hbm.at[ig.at[j]],
                                  ps_v.at[pl.ds(j * _SUB, _SUB)], s_in).wait()
            pltpu.make_async_copy(p_hbm.at[ig.at[_NSUB + j]],
                                  pd_v.at[pl.ds(j * _SUB, _SUB)], s_in).wait()

    def issue_scatters(ph):
        _, isx, _, _, _, _, cs_v, cd_v, _, _, _, s_sc = bufs[ph]
        for j in range(_NSUB):
            pltpu.async_copy(cs_v.at[pl.ds(j * _SUB, _SUB)],
                             acc.at[isx.at[j]], s_sc, add=True)
            pltpu.async_copy(cd_v.at[pl.ds(j * _SUB, _SUB)],
                             acc.at[isx.at[_NSUB + j]], s_sc, add=True)

    def wait_scatters(ph):
        _, isx, _, _, _, _, cs_v, cd_v, _, _, _, s_sc = bufs[ph]
        for j in range(_NSUB):
            pltpu.make_async_copy(cs_v.at[pl.ds(j * _SUB, _SUB)],
                                  acc.at[isx.at[j]], s_sc).wait()
            pltpu.make_async_copy(cd_v.at[pl.ds(j * _SUB, _SUB)],
                                  acc.at[isx.at[_NSUB + j]], s_sc).wait()

    def issue_idx(q, ph, which):
        ig, isx, _, _, _, _, _, _, s_ig, s_is, _, _ = bufs[ph]
        ref = ig if which == 0 else isx
        sem = s_ig if which == 0 else s_is
        irow = ir0 + q * _NSUB
        pltpu.async_copy(srcr.at[pl.ds(irow, _NSUB)],
                         ref.at[pl.ds(0, _NSUB)], sem)
        pltpu.async_copy(dstr.at[pl.ds(irow, _NSUB)],
                         ref.at[pl.ds(_NSUB, _NSUB)], sem)

    def wait_idx(ph, which):
        ig, isx, _, _, _, _, _, _, s_ig, s_is, _, _ = bufs[ph]
        ref = ig if which == 0 else isx
        sem = s_ig if which == 0 else s_is
        pltpu.make_async_copy(srcr.at[pl.ds(0, _NSUB)],
                              ref.at[pl.ds(0, _NSUB)], sem).wait()
        pltpu.make_async_copy(dstr.at[pl.ds(0, _NSUB)],
                              ref.at[pl.ds(_NSUB, _NSUB)], sem).wait()

    def compute(ph):
        _, _, rs_v, rd_v, ps_v, pd_v, cs_v, cd_v, _, _, _, _ = bufs[ph]
        _compute_chunk(rs_v, rd_v, ps_v, pd_v, cs_v, cd_v, iota)

    def phase(kk, q, ph):
        @pl.when(kk > 0)
        def _():
            wait_scatters(ph)
        issue_idx(q, ph, 1)
        wait_inputs(ph)
        issue_idx(jnp.minimum(q + 2, last), ph, 0)
        compute(ph)
        wait_idx(ph, 1)
        issue_scatters(ph)
        wait_idx(ph, 0)
        issue_inputs(jnp.minimum(q + 2, last), ph)

    issue_idx(0, 0, 0)
    wait_idx(0, 0)
    issue_inputs(0, 0)
    issue_idx(1, 1, 0)
    wait_idx(1, 0)
    issue_inputs(1, 1)

    def pair_body(kk, carry):
        phase(kk, 2 * kk, 0)
        phase(kk, 2 * kk + 1, 1)
        return carry

    lax.fori_loop(0, _PAIRS, pair_body, 0)

    q = last
    wait_scatters(0)
    issue_idx(q, 0, 1)
    wait_inputs(0)
    compute(0)
    wait_idx(0, 1)
    issue_scatters(0)
    wait_inputs(1)
    wait_scatters(0)
    wait_scatters(1)

    plsc.subcore_barrier()

    @pl.when(s < _NSTAGE)
    def _():
        def wb_body(j, carry):
            rr = r0 + j * _SC
            pltpu.sync_copy(acc.at[pl.ds(rr, _SC)], csA)
            pltpu.sync_copy(csA, out_hbm.at[pl.ds(c * _N + rr, _SC)])
            return carry

        lax.fori_loop(0, _RPT // _SC, wb_body, 0)


_sheaf_step = functools.partial(
    pl.kernel,
    out_type=jax.ShapeDtypeStruct((_NC * _N, _BD), jnp.float32),
    mesh=plsc.VectorSubcoreMesh(core_axis_name="c", subcore_axis_name="s"),
    scratch_types=[
        pltpu.VMEM_SHARED((_N, _BD), jnp.float32),
        pltpu.VMEM((2 * _NSUB, _SUB), jnp.int32),
        pltpu.VMEM((2 * _NSUB, _SUB), jnp.int32),
        pltpu.VMEM((2 * _NSUB, _SUB), jnp.int32),
        pltpu.VMEM((2 * _NSUB, _SUB), jnp.int32),
        pltpu.VMEM((_C, 4, 4), jnp.float32),
        pltpu.VMEM((_C, 4, 4), jnp.float32),
        pltpu.VMEM((_C, 4, 4), jnp.float32),
        pltpu.VMEM((_C, 4, 4), jnp.float32),
        pltpu.VMEM((_C, _BD), jnp.float32),
        pltpu.VMEM((_C, _BD), jnp.float32),
        pltpu.VMEM((_C, _BD), jnp.float32),
        pltpu.VMEM((_C, _BD), jnp.float32),
        pltpu.VMEM((_C, _BD), jnp.float32),
        pltpu.VMEM((_C, _BD), jnp.float32),
        pltpu.VMEM((_C, _BD), jnp.float32),
        pltpu.VMEM((_C, _BD), jnp.float32),
        pltpu.SemaphoreType.DMA,
        pltpu.SemaphoreType.DMA,
        pltpu.SemaphoreType.DMA,
        pltpu.SemaphoreType.DMA,
        pltpu.SemaphoreType.DMA,
        pltpu.SemaphoreType.DMA,
        pltpu.SemaphoreType.DMA,
        pltpu.SemaphoreType.DMA,
    ],
    compiler_params=pltpu.CompilerParams(
        needs_layout_passes=False, use_tc_tiling_on_sc=False),
)(_sheaf_step_body)


def kernel(c0, src, dst, R_src, R_dst, poly_coeffs):
    B, N, D = c0.shape
    E = src.shape[0]
    p = jnp.transpose(c0, (1, 0, 2)).reshape(N, B * D)
    srcr = src.astype(jnp.int32).reshape(E // _SUB, _SUB)
    dstr = dst.astype(jnp.int32).reshape(E // _SUB, _SUB)
    zero = jnp.zeros((N, B * D), jnp.float32)

    out = poly_coeffs[0] * p
    v = p
    for k in range(1, 4):
        parts = _sheaf_step(v, srcr, dstr, R_src, R_dst, zero)
        v = parts[:N] + parts[N:]
        out = out + poly_coeffs[k] * v
    return out.reshape(N, B, D).transpose(1, 0, 2)


# speedup vs baseline: 1.2861x; 1.2861x over previous
import functools

import jax
import jax.numpy as jnp
from jax import lax
from jax.experimental import pallas as pl
from jax.experimental.pallas import tpu as pltpu
from jax.experimental.pallas import tpu_sc as plsc

_N = 100000
_E = 1600000
_BD = 8
_RF = 16
_NC = 2
_NS = 16
_NW = _NC * _NS
_EPT = _E // _NW
_C = 400
_NCHUNK = _EPT // _C
_PAIRS = (_NCHUNK - 1) // 2
_G = _C // 16
_SUB = 50
_NSUB = _C // _SUB
_IPT = _EPT // _SUB
_NSTAGE = 10
_RPT = _N // _NSTAGE
_SC = 400


def _compute_chunk(rs_v, rd_v, ps_v, pd_v, cs_v, cd_v, iota):
    def group_body(g, carry):
        rows = g * 16 + iota
        cols = [jnp.full((16,), j, jnp.int32) for j in range(16)]
        rflat = (g * 256 + iota * 16)
        Rs = [plsc.load_gather(rs_v, [rflat + j]) for j in range(16)]
        Rd = [plsc.load_gather(rd_v, [rflat + j]) for j in range(16)]
        Ps = [plsc.load_gather(ps_v, [rows, cols[j]]) for j in range(8)]
        Pd = [plsc.load_gather(pd_v, [rows, cols[j]]) for j in range(8)]
        for b in range(2):
            r = []
            for a in range(4):
                t = Rs[a * 4] * Ps[b * 4]
                u = Rd[a * 4] * Pd[b * 4]
                for d in range(1, 4):
                    t = t + Rs[a * 4 + d] * Ps[b * 4 + d]
                    u = u + Rd[a * 4 + d] * Pd[b * 4 + d]
                r.append(t - u)
            for d in range(4):
                cs = Rs[d] * r[0]
                cd = Rd[d] * r[0]
                for a in range(1, 4):
                    cs = cs + Rs[a * 4 + d] * r[a]
                    cd = cd + Rd[a * 4 + d] * r[a]
                plsc.store_scatter(cs_v, [rows, cols[b * 4 + d]], cs)
                plsc.store_scatter(cd_v, [rows, cols[b * 4 + d]], -cd)
        return carry

    lax.fori_loop(0, _G, group_body, 0)


def _sheaf_step_body(p_hbm, srcr, dstr, rs_hbm, rd_hbm, zero_hbm, out_hbm,
                     acc,
                     igA, igB, isA, isB,
                     rsA, rdA, rsB, rdB,
                     psA, pdA, psB, pdB,
                     csA, cdA, csB, cdB,
                     s_igA, s_igB, s_isA, s_isB,
                     s_inA, s_inB, s_scA, s_scB):
    c = lax.axis_index("c")
    s = lax.axis_index("s")
    w = c * _NS + s

    r0 = s * _RPT

    @pl.when(s < _NSTAGE)
    def _():
        def zero_body(j, carry):
            rr = r0 + j * _SC
            pltpu.sync_copy(zero_hbm.at[pl.ds(rr, _SC)], csA)
            pltpu.sync_copy(csA, acc.at[pl.ds(rr, _SC)])
            return carry

        lax.fori_loop(0, _RPT // _SC, zero_body, 0)

    plsc.subcore_barrier()

    iota = lax.iota(jnp.int32, 16)
    e0 = w * _EPT
    ir0 = w * _IPT
    last = _NCHUNK - 1

    bufs = {
        0: (igA, isA, rsA, rdA, psA, pdA, csA, cdA, s_igA, s_isA, s_inA, s_scA),
        1: (igB, isB, rsB, rdB, psB, pdB, csB, cdB, s_igB, s_isB, s_inB, s_scB),
    }

    def issue_inputs(q, ph):
        ig, _, rs_v, rd_v, ps_v, pd_v, _, _, _, _, s_in, _ = bufs[ph]
        eofs = (e0 + q * _C) * _RF
        pltpu.async_copy(rs_hbm.at[pl.ds(eofs, _C * _RF)], rs_v, s_in)
        pltpu.async_copy(rd_hbm.at[pl.ds(eofs, _C * _RF)], rd_v, s_in)
        for j in range(_NSUB):
            pltpu.async_copy(p_hbm.at[ig.at[j]],
                             ps_v.at[pl.ds(j * _SUB, _SUB)], s_in)
            pltpu.async_copy(p_hbm.at[ig.at[_NSUB + j]],
                             pd_v.at[pl.ds(j * _SUB, _SUB)], s_in)

    def wait_inputs(ph):
        ig, _, rs_v, rd_v, ps_v, pd_v, _, _, _, _, s_in, _ = bufs[ph]
        pltpu.make_async_copy(rs_hbm.at[pl.ds(0, _C * _RF)], rs_v, s_in).wait()
        pltpu.make_async_copy(rd_hbm.at[pl.ds(0, _C * _RF)], rd_v, s_in).wait()
        for j in range(_NSUB):
            pltpu.make_async_copy(p_hbm.at[ig.at[j]],
                                  ps_v.at[pl.ds(j * _SUB, _SUB)], s_in).wait()
            pltpu.make_async_copy(p_hbm.at[ig.at[_NSUB + j]],
                                  pd_v.at[pl.ds(j * _SUB, _SUB)], s_in).wait()

    def issue_scatters(ph):
        _, isx, _, _, _, _, cs_v, cd_v, _, _, _, s_sc = bufs[ph]
        for j in range(_NSUB):
            pltpu.async_copy(cs_v.at[pl.ds(j * _SUB, _SUB)],
                             acc.at[isx.at[j]], s_sc, add=True)
            pltpu.async_copy(cd_v.at[pl.ds(j * _SUB, _SUB)],
                             acc.at[isx.at[_NSUB + j]], s_sc, add=True)

    def wait_scatters(ph):
        _, isx, _, _, _, _, cs_v, cd_v, _, _, _, s_sc = bufs[ph]
        for j in range(_NSUB):
            pltpu.make_async_copy(cs_v.at[pl.ds(j * _SUB, _SUB)],
                                  acc.at[isx.at[j]], s_sc).wait()
            pltpu.make_async_copy(cd_v.at[pl.ds(j * _SUB, _SUB)],
                                  acc.at[isx.at[_NSUB + j]], s_sc).wait()

    def issue_idx(q, ph, which):
        ig, isx, _, _, _, _, _, _, s_ig, s_is, _, _ = bufs[ph]
        ref = ig if which == 0 else isx
        sem = s_ig if which == 0 else s_is
        irow = ir0 + q * _NSUB
        pltpu.async_copy(srcr.at[pl.ds(irow, _NSUB)],
                         ref.at[pl.ds(0, _NSUB)], sem)
        pltpu.async_copy(dstr.at[pl.ds(irow, _NSUB)],
                         ref.at[pl.ds(_NSUB, _NSUB)], sem)

    def wait_idx(ph, which):
        ig, isx, _, _, _, _, _, _, s_ig, s_is, _, _ = bufs[ph]
        ref = ig if which == 0 else isx
        sem = s_ig if which == 0 else s_is
        pltpu.make_async_copy(srcr.at[pl.ds(0, _NSUB)],
                              ref.at[pl.ds(0, _NSUB)], sem).wait()
        pltpu.make_async_copy(dstr.at[pl.ds(0, _NSUB)],
                              ref.at[pl.ds(_NSUB, _NSUB)], sem).wait()

    def compute(ph):
        _, _, rs_v, rd_v, ps_v, pd_v, cs_v, cd_v, _, _, _, _ = bufs[ph]
        _compute_chunk(rs_v, rd_v, ps_v, pd_v, cs_v, cd_v, iota)

    def phase(kk, q, ph):
        @pl.when(kk > 0)
        def _():
            wait_scatters(ph)
        issue_idx(q, ph, 1)
        wait_inputs(ph)
        issue_idx(jnp.minimum(q + 2, last), ph, 0)
        compute(ph)
        wait_idx(ph, 1)
        issue_scatters(ph)
        wait_idx(ph, 0)
        issue_inputs(jnp.minimum(q + 2, last), ph)

    issue_idx(0, 0, 0)
    wait_idx(0, 0)
    issue_inputs(0, 0)
    issue_idx(1, 1, 0)
    wait_idx(1, 0)
    issue_inputs(1, 1)

    def pair_body(kk, carry):
        phase(kk, 2 * kk, 0)
        phase(kk, 2 * kk + 1, 1)
        return carry

    lax.fori_loop(0, _PAIRS, pair_body, 0)

    q = last
    wait_scatters(0)
    issue_idx(q, 0, 1)
    wait_inputs(0)
    compute(0)
    wait_idx(0, 1)
    issue_scatters(0)
    wait_inputs(1)
    wait_scatters(0)
    wait_scatters(1)

    plsc.subcore_barrier()

    @pl.when(s < _NSTAGE)
    def _():
        def wb_body(j, carry):
            rr = r0 + j * _SC
            pltpu.sync_copy(acc.at[pl.ds(rr, _SC)], csA)
            pltpu.sync_copy(csA, out_hbm.at[pl.ds(c * _N + rr, _SC)])
            return carry

        lax.fori_loop(0, _RPT // _SC, wb_body, 0)


_sheaf_step = functools.partial(
    pl.kernel,
    out_type=jax.ShapeDtypeStruct((_NC * _N, _BD), jnp.float32),
    mesh=plsc.VectorSubcoreMesh(core_axis_name="c", subcore_axis_name="s"),
    scratch_types=[
        pltpu.VMEM_SHARED((_N, _BD), jnp.float32),
        pltpu.VMEM((2 * _NSUB, _SUB), jnp.int32),
        pltpu.VMEM((2 * _NSUB, _SUB), jnp.int32),
        pltpu.VMEM((2 * _NSUB, _SUB), jnp.int32),
        pltpu.VMEM((2 * _NSUB, _SUB), jnp.int32),
        pltpu.VMEM((_C * _RF,), jnp.float32),
        pltpu.VMEM((_C * _RF,), jnp.float32),
        pltpu.VMEM((_C * _RF,), jnp.float32),
        pltpu.VMEM((_C * _RF,), jnp.float32),
        pltpu.VMEM((_C, _BD), jnp.float32),
        pltpu.VMEM((_C, _BD), jnp.float32),
        pltpu.VMEM((_C, _BD), jnp.float32),
        pltpu.VMEM((_C, _BD), jnp.float32),
        pltpu.VMEM((_C, _BD), jnp.float32),
        pltpu.VMEM((_C, _BD), jnp.float32),
        pltpu.VMEM((_C, _BD), jnp.float32),
        pltpu.VMEM((_C, _BD), jnp.float32),
        pltpu.SemaphoreType.DMA,
        pltpu.SemaphoreType.DMA,
        pltpu.SemaphoreType.DMA,
        pltpu.SemaphoreType.DMA,
        pltpu.SemaphoreType.DMA,
        pltpu.SemaphoreType.DMA,
        pltpu.SemaphoreType.DMA,
        pltpu.SemaphoreType.DMA,
    ],
    compiler_params=pltpu.CompilerParams(
        needs_layout_passes=False, use_tc_tiling_on_sc=False),
)(_sheaf_step_body)


def kernel(c0, src, dst, R_src, R_dst, poly_coeffs):
    B, N, D = c0.shape
    E = src.shape[0]
    p = jnp.transpose(c0, (1, 0, 2)).reshape(N, B * D)
    rs = R_src.reshape(E * _RF)
    rd = R_dst.reshape(E * _RF)
    srcr = src.astype(jnp.int32).reshape(E // _SUB, _SUB)
    dstr = dst.astype(jnp.int32).reshape(E // _SUB, _SUB)
    zero = jnp.zeros((N, B * D), jnp.float32)

    out = poly_coeffs[0] * p
    v = p
    for k in range(1, 4):
        parts = _sheaf_step(v, srcr, dstr, rs, rd, zero)
        v = parts[:N] + parts[N:]
        out = out + poly_coeffs[k] * v
    return out.reshape(N, B, D).transpose(1, 0, 2)


# speedup vs baseline: 4.0295x; 3.1331x over previous
import functools

import jax
import jax.numpy as jnp
from jax import lax
from jax.experimental import pallas as pl
from jax.experimental.pallas import tpu as pltpu
from jax.experimental.pallas import tpu_sc as plsc

_N = 100000
_E = 1600000
_BD = 8
_RF = 16
_NC = 2
_NS = 16
_NW = _NC * _NS
_EPT = _E // _NW
_C = 400
_NCHUNK = _EPT // _C
_PAIRS = (_NCHUNK - 1) // 2
_G = _C // 16
_SUB = 50
_NSUB = _C // _SUB
_IPT = _EPT // _SUB
_NSTAGE = 10
_RPT = _N // _NSTAGE
_SC = 400


def _compute_chunk(rs_v, rd_v, ps_v, pd_v, cs_v, cd_v, iota):
    def group_body(g, carry):
        rows = g * 16 + iota
        cols = [jnp.full((16,), j, jnp.int32) for j in range(16)]
        Rs = [rs_v[j, pl.ds(g * 16, 16)] for j in range(_RF)]
        Rd = [rd_v[j, pl.ds(g * 16, 16)] for j in range(_RF)]
        Ps = [plsc.load_gather(ps_v, [rows, cols[j]]) for j in range(8)]
        Pd = [plsc.load_gather(pd_v, [rows, cols[j]]) for j in range(8)]
        for b in range(2):
            r = []
            for a in range(4):
                t = Rs[a * 4] * Ps[b * 4]
                u = Rd[a * 4] * Pd[b * 4]
                for d in range(1, 4):
                    t = t + Rs[a * 4 + d] * Ps[b * 4 + d]
                    u = u + Rd[a * 4 + d] * Pd[b * 4 + d]
                r.append(t - u)
            for d in range(4):
                cs = Rs[d] * r[0]
                cd = Rd[d] * r[0]
                for a in range(1, 4):
                    cs = cs + Rs[a * 4 + d] * r[a]
                    cd = cd + Rd[a * 4 + d] * r[a]
                plsc.store_scatter(cs_v, [rows, cols[b * 4 + d]], cs)
                plsc.store_scatter(cd_v, [rows, cols[b * 4 + d]], -cd)
        return carry

    lax.fori_loop(0, _G, group_body, 0)


def _sheaf_step_body(p_hbm, srcr, dstr, rs_hbm, rd_hbm, zero_hbm, out_hbm,
                     acc,
                     igA, igB, isA, isB,
                     rsA, rdA, rsB, rdB,
                     psA, pdA, psB, pdB,
                     csA, cdA, csB, cdB,
                     s_igA, s_igB, s_isA, s_isB,
                     s_inA, s_inB, s_scA, s_scB):
    c = lax.axis_index("c")
    s = lax.axis_index("s")
    w = c * _NS + s

    r0 = s * _RPT

    @pl.when(s < _NSTAGE)
    def _():
        def zero_body(j, carry):
            rr = r0 + j * _SC
            pltpu.sync_copy(zero_hbm.at[pl.ds(rr, _SC)], csA)
            pltpu.sync_copy(csA, acc.at[pl.ds(rr, _SC)])
            return carry

        lax.fori_loop(0, _RPT // _SC, zero_body, 0)

    plsc.subcore_barrier()

    iota = lax.iota(jnp.int32, 16)
    e0 = w * _EPT
    ir0 = w * _IPT
    last = _NCHUNK - 1

    bufs = {
        0: (igA, isA, rsA, rdA, psA, pdA, csA, cdA, s_igA, s_isA, s_inA, s_scA),
        1: (igB, isB, rsB, rdB, psB, pdB, csB, cdB, s_igB, s_isB, s_inB, s_scB),
    }

    def issue_inputs(q, ph):
        ig, _, rs_v, rd_v, ps_v, pd_v, _, _, _, _, s_in, _ = bufs[ph]
        erow = e0 + q * _C
        for j in range(_RF):
            pltpu.async_copy(rs_hbm.at[j, pl.ds(erow, _C)], rs_v.at[j], s_in)
            pltpu.async_copy(rd_hbm.at[j, pl.ds(erow, _C)], rd_v.at[j], s_in)
        for j in range(_NSUB):
            pltpu.async_copy(p_hbm.at[ig.at[j]],
                             ps_v.at[pl.ds(j * _SUB, _SUB)], s_in)
            pltpu.async_copy(p_hbm.at[ig.at[_NSUB + j]],
                             pd_v.at[pl.ds(j * _SUB, _SUB)], s_in)

    def wait_inputs(ph):
        ig, _, rs_v, rd_v, ps_v, pd_v, _, _, _, _, s_in, _ = bufs[ph]
        for j in range(_RF):
            pltpu.make_async_copy(rs_hbm.at[j, pl.ds(0, _C)],
                                  rs_v.at[j], s_in).wait()
            pltpu.make_async_copy(rd_hbm.at[j, pl.ds(0, _C)],
                                  rd_v.at[j], s_in).wait()
        for j in range(_NSUB):
            pltpu.make_async_copy(p_hbm.at[ig.at[j]],
                                  ps_v.at[pl.ds(j * _SUB, _SUB)], s_in).wait()
            pltpu.make_async_copy(p_hbm.at[ig.at[_NSUB + j]],
                                  pd_v.at[pl.ds(j * _SUB, _SUB)], s_in).wait()

    def issue_scatters(ph):
        _, isx, _, _, _, _, cs_v, cd_v, _, _, _, s_sc = bufs[ph]
        for j in range(_NSUB):
            pltpu.async_copy(cs_v.at[pl.ds(j * _SUB, _SUB)],
                             acc.at[isx.at[j]], s_sc, add=True)
            pltpu.async_copy(cd_v.at[pl.ds(j * _SUB, _SUB)],
                             acc.at[isx.at[_NSUB + j]], s_sc, add=True)

    def wait_scatters(ph):
        _, isx, _, _, _, _, cs_v, cd_v, _, _, _, s_sc = bufs[ph]
        for j in range(_NSUB):
            pltpu.make_async_copy(cs_v.at[pl.ds(j * _SUB, _SUB)],
                                  acc.at[isx.at[j]], s_sc).wait()
            pltpu.make_async_copy(cd_v.at[pl.ds(j * _SUB, _SUB)],
                                  acc.at[isx.at[_NSUB + j]], s_sc).wait()

    def issue_idx(q, ph, which):
        ig, isx, _, _, _, _, _, _, s_ig, s_is, _, _ = bufs[ph]
        ref = ig if which == 0 else isx
        sem = s_ig if which == 0 else s_is
        irow = ir0 + q * _NSUB
        pltpu.async_copy(srcr.at[pl.ds(irow, _NSUB)],
                         ref.at[pl.ds(0, _NSUB)], sem)
        pltpu.async_copy(dstr.at[pl.ds(irow, _NSUB)],
                         ref.at[pl.ds(_NSUB, _NSUB)], sem)

    def wait_idx(ph, which):
        ig, isx, _, _, _, _, _, _, s_ig, s_is, _, _ = bufs[ph]
        ref = ig if which == 0 else isx
        sem = s_ig if which == 0 else s_is
        pltpu.make_async_copy(srcr.at[pl.ds(0, _NSUB)],
                              ref.at[pl.ds(0, _NSUB)], sem).wait()
        pltpu.make_async_copy(dstr.at[pl.ds(0, _NSUB)],
                              ref.at[pl.ds(_NSUB, _NSUB)], sem).wait()

    def compute(ph):
        _, _, rs_v, rd_v, ps_v, pd_v, cs_v, cd_v, _, _, _, _ = bufs[ph]
        _compute_chunk(rs_v, rd_v, ps_v, pd_v, cs_v, cd_v, iota)

    def phase(kk, q, ph):
        @pl.when(kk > 0)
        def _():
            wait_scatters(ph)
        issue_idx(q, ph, 1)
        wait_inputs(ph)
        issue_idx(jnp.minimum(q + 2, last), ph, 0)
        compute(ph)
        wait_idx(ph, 1)
        issue_scatters(ph)
        wait_idx(ph, 0)
        issue_inputs(jnp.minimum(q + 2, last), ph)

    issue_idx(0, 0, 0)
    wait_idx(0, 0)
    issue_inputs(0, 0)
    issue_idx(1, 1, 0)
    wait_idx(1, 0)
    issue_inputs(1, 1)

    def pair_body(kk, carry):
        phase(kk, 2 * kk, 0)
        phase(kk, 2 * kk + 1, 1)
        return carry

    lax.fori_loop(0, _PAIRS, pair_body, 0)

    q = last
    wait_scatters(0)
    issue_idx(q, 0, 1)
    wait_inputs(0)
    compute(0)
    wait_idx(0, 1)
    issue_scatters(0)
    wait_inputs(1)
    wait_scatters(0)
    wait_scatters(1)

    plsc.subcore_barrier()

    @pl.when(s < _NSTAGE)
    def _():
        def wb_body(j, carry):
            rr = r0 + j * _SC
            pltpu.sync_copy(acc.at[pl.ds(rr, _SC)], csA)
            pltpu.sync_copy(csA, out_hbm.at[pl.ds(c * _N + rr, _SC)])
            return carry

        lax.fori_loop(0, _RPT // _SC, wb_body, 0)


_sheaf_step = functools.partial(
    pl.kernel,
    out_type=jax.ShapeDtypeStruct((_NC * _N, _BD), jnp.float32),
    mesh=plsc.VectorSubcoreMesh(core_axis_name="c", subcore_axis_name="s"),
    scratch_types=[
        pltpu.VMEM_SHARED((_N, _BD), jnp.float32),
        pltpu.VMEM((2 * _NSUB, _SUB), jnp.int32),
        pltpu.VMEM((2 * _NSUB, _SUB), jnp.int32),
        pltpu.VMEM((2 * _NSUB, _SUB), jnp.int32),
        pltpu.VMEM((2 * _NSUB, _SUB), jnp.int32),
        pltpu.VMEM((_RF, _C), jnp.float32),
        pltpu.VMEM((_RF, _C), jnp.float32),
        pltpu.VMEM((_RF, _C), jnp.float32),
        pltpu.VMEM((_RF, _C), jnp.float32),
        pltpu.VMEM((_C, _BD), jnp.float32),
        pltpu.VMEM((_C, _BD), jnp.float32),
        pltpu.VMEM((_C, _BD), jnp.float32),
        pltpu.VMEM((_C, _BD), jnp.float32),
        pltpu.VMEM((_C, _BD), jnp.float32),
        pltpu.VMEM((_C, _BD), jnp.float32),
        pltpu.VMEM((_C, _BD), jnp.float32),
        pltpu.VMEM((_C, _BD), jnp.float32),
        pltpu.SemaphoreType.DMA,
        pltpu.SemaphoreType.DMA,
        pltpu.SemaphoreType.DMA,
        pltpu.SemaphoreType.DMA,
        pltpu.SemaphoreType.DMA,
        pltpu.SemaphoreType.DMA,
        pltpu.SemaphoreType.DMA,
        pltpu.SemaphoreType.DMA,
    ],
    compiler_params=pltpu.CompilerParams(
        needs_layout_passes=False, use_tc_tiling_on_sc=False),
)(_sheaf_step_body)


def kernel(c0, src, dst, R_src, R_dst, poly_coeffs):
    B, N, D = c0.shape
    E = src.shape[0]
    p = jnp.transpose(c0, (1, 0, 2)).reshape(N, B * D)
    rs = jnp.transpose(R_src.reshape(E, _RF))
    rd = jnp.transpose(R_dst.reshape(E, _RF))
    srcr = src.astype(jnp.int32).reshape(E // _SUB, _SUB)
    dstr = dst.astype(jnp.int32).reshape(E // _SUB, _SUB)
    zero = jnp.zeros((N, B * D), jnp.float32)

    out = poly_coeffs[0] * p
    v = p
    for k in range(1, 4):
        parts = _sheaf_step(v, srcr, dstr, rs, rd, zero)
        v = parts[:N] + parts[N:]
        out = out + poly_coeffs[k] * v
    return out.reshape(N, B, D).transpose(1, 0, 2)


# speedup vs baseline: 10.8070x; 2.6819x over previous
import functools

import jax
import jax.numpy as jnp
from jax import lax
from jax.experimental import pallas as pl
from jax.experimental.pallas import tpu as pltpu
from jax.experimental.pallas import tpu_sc as plsc

_N = 100000
_E = 1600000
_BD = 8
_RF = 16
_NC = 2
_NS = 16
_NW = _NC * _NS
_EBLK = 128
_NB = _E // _EBLK
_CB = 4
_C = _CB * _EBLK
_TCH = _NB // _CB
_BASE = _TCH // _NW
_EXTRA = _TCH - _BASE * _NW
_NTRIP = _BASE + 1
_PAIRS = _NTRIP // 2
_G = _C // 16
_PLANE = _E * 4
_NSTAGE = 10
_RPT = _N // _NSTAGE
_SC = 400


def _compute_chunk(rs_v, rd_v, ps_v, pd_v, cs_v, cd_v, iota):
    def group_body(g, carry):
        rows = g * 16 + iota
        cols = [jnp.full((16,), j, jnp.int32) for j in range(_BD)]
        rbase = (g // 8) * 512 + (g % 8) * 16
        Rs = [rs_v[a, pl.ds(rbase + d * 128, 16)]
              for a in range(4) for d in range(4)]
        Rd = [rd_v[a, pl.ds(rbase + d * 128, 16)]
              for a in range(4) for d in range(4)]
        Ps = [plsc.load_gather(ps_v, [rows, cols[j]]) for j in range(_BD)]
        Pd = [plsc.load_gather(pd_v, [rows, cols[j]]) for j in range(_BD)]
        for b in range(2):
            r = []
            for a in range(4):
                t = Rs[a * 4] * Ps[b * 4]
                u = Rd[a * 4] * Pd[b * 4]
                for d in range(1, 4):
                    t = t + Rs[a * 4 + d] * Ps[b * 4 + d]
                    u = u + Rd[a * 4 + d] * Pd[b * 4 + d]
                r.append(t - u)
            for d in range(4):
                cs = Rs[d] * r[0]
                cd = Rd[d] * r[0]
                for a in range(1, 4):
                    cs = cs + Rs[a * 4 + d] * r[a]
                    cd = cd + Rd[a * 4 + d] * r[a]
                plsc.store_scatter(cs_v, [rows, cols[b * 4 + d]], cs)
                plsc.store_scatter(cd_v, [rows, cols[b * 4 + d]], -cd)
        return carry

    lax.fori_loop(0, _G, group_body, 0)


def _sheaf_step_body(p_hbm, srcr, dstr, rs_hbm, rd_hbm, zero_hbm, out_hbm,
                     acc,
                     igA, igB, isA, isB,
                     rsA, rdA, rsB, rdB,
                     psA, pdA, psB, pdB,
                     csA, cdA, csB, cdB,
                     s_igA, s_igB, s_isA, s_isB,
                     s_inA, s_inB, s_scA, s_scB):
    c = lax.axis_index("c")
    s = lax.axis_index("s")
    w = c * _NS + s

    r0 = s * _RPT

    @pl.when(s < _NSTAGE)
    def _():
        def zero_body(j, carry):
            rr = r0 + j * _SC
            pltpu.sync_copy(zero_hbm.at[pl.ds(rr, _SC)], csA.at[pl.ds(0, _SC)])
            pltpu.sync_copy(csA.at[pl.ds(0, _SC)], acc.at[pl.ds(rr, _SC)])
            return carry

        lax.fori_loop(0, _RPT // _SC, zero_body, 0)

    plsc.subcore_barrier()

    iota = lax.iota(jnp.int32, 16)
    nch = _BASE + (w < _EXTRA).astype(jnp.int32)
    ch0 = _BASE * w + jnp.minimum(w, _EXTRA)

    bufs = {
        0: (igA, isA, rsA, rdA, psA, pdA, csA, cdA, s_igA, s_isA, s_inA, s_scA),
        1: (igB, isB, rsB, rdB, psB, pdB, csB, cdB, s_igB, s_isB, s_inB, s_scB),
    }

    def chunk_of(i):
        return ch0 + jnp.minimum(i, nch - 1)

    def issue_inputs(i, ph):
        ig, _, rs_v, rd_v, ps_v, pd_v, _, _, _, _, s_in, _ = bufs[ph]
        r = chunk_of(i)
        for a in range(4):
            ofs = a * _PLANE + r * 2048
            pltpu.async_copy(rs_hbm.at[pl.ds(ofs, 2048)], rs_v.at[a], s_in)
            pltpu.async_copy(rd_hbm.at[pl.ds(ofs, 2048)], rd_v.at[a], s_in)
        for j in range(_CB):
            pltpu.async_copy(p_hbm.at[ig.at[j]],
                             ps_v.at[pl.ds(j * _EBLK, _EBLK)], s_in)
            pltpu.async_copy(p_hbm.at[ig.at[_CB + j]],
                             pd_v.at[pl.ds(j * _EBLK, _EBLK)], s_in)

    def wait_inputs(ph):
        ig, _, rs_v, rd_v, ps_v, pd_v, _, _, _, _, s_in, _ = bufs[ph]
        for a in range(4):
            pltpu.make_async_copy(rs_hbm.at[pl.ds(0, 2048)],
                                  rs_v.at[a], s_in).wait()
            pltpu.make_async_copy(rd_hbm.at[pl.ds(0, 2048)],
                                  rd_v.at[a], s_in).wait()
        for j in range(_CB):
            pltpu.make_async_copy(p_hbm.at[ig.at[j]],
                                  ps_v.at[pl.ds(j * _EBLK, _EBLK)], s_in).wait()
            pltpu.make_async_copy(p_hbm.at[ig.at[_CB + j]],
                                  pd_v.at[pl.ds(j * _EBLK, _EBLK)], s_in).wait()

    def issue_scatters(ph):
        _, isx, _, _, _, _, cs_v, cd_v, _, _, _, s_sc = bufs[ph]
        for j in range(_CB):
            pltpu.async_copy(cs_v.at[pl.ds(j * _EBLK, _EBLK)],
                             acc.at[isx.at[j]], s_sc, add=True)
            pltpu.async_copy(cd_v.at[pl.ds(j * _EBLK, _EBLK)],
                             acc.at[isx.at[_CB + j]], s_sc, add=True)

    def wait_scatters(ph):
        _, isx, _, _, _, _, cs_v, cd_v, _, _, _, s_sc = bufs[ph]
        for j in range(_CB):
            pltpu.make_async_copy(cs_v.at[pl.ds(j * _EBLK, _EBLK)],
                                  acc.at[isx.at[j]], s_sc).wait()
            pltpu.make_async_copy(cd_v.at[pl.ds(j * _EBLK, _EBLK)],
                                  acc.at[isx.at[_CB + j]], s_sc).wait()

    def issue_idx(i, ph, which):
        ig, isx, _, _, _, _, _, _, s_ig, s_is, _, _ = bufs[ph]
        ref = ig if which == 0 else isx
        sem = s_ig if which == 0 else s_is
        irow = chunk_of(i) * _CB
        pltpu.async_copy(srcr.at[pl.ds(irow, _CB)],
                         ref.at[pl.ds(0, _CB)], sem)
        pltpu.async_copy(dstr.at[pl.ds(irow, _CB)],
                         ref.at[pl.ds(_CB, _CB)], sem)

    def wait_idx(ph, which):
        ig, isx, _, _, _, _, _, _, s_ig, s_is, _, _ = bufs[ph]
        ref = ig if which == 0 else isx
        sem = s_ig if which == 0 else s_is
        pltpu.make_async_copy(srcr.at[pl.ds(0, _CB)],
                              ref.at[pl.ds(0, _CB)], sem).wait()
        pltpu.make_async_copy(dstr.at[pl.ds(0, _CB)],
                              ref.at[pl.ds(_CB, _CB)], sem).wait()

    def compute(ph):
        _, _, rs_v, rd_v, ps_v, pd_v, cs_v, cd_v, _, _, _, _ = bufs[ph]
        _compute_chunk(rs_v, rd_v, ps_v, pd_v, cs_v, cd_v, iota)

    def phase(kk, i, ph):
        @pl.when(kk > 0)
        def _():
            wait_scatters(ph)
        issue_idx(i, ph, 1)
        wait_inputs(ph)
        issue_idx(i + 2, ph, 0)
        compute(ph)
        wait_idx(ph, 1)

        @pl.when(i < nch)
        def _():
            issue_scatters(ph)

        wait_idx(ph, 0)
        issue_inputs(i + 2, ph)

    issue_idx(0, 0, 0)
    wait_idx(0, 0)
    issue_inputs(0, 0)
    issue_idx(1, 1, 0)
    wait_idx(1, 0)
    issue_inputs(1, 1)

    def pair_body(kk, carry):
        phase(kk, 2 * kk, 0)
        phase(kk, 2 * kk + 1, 1)
        return carry

    lax.fori_loop(0, _PAIRS, pair_body, 0)

    wait_inputs(0)
    wait_inputs(1)
    wait_scatters(0)

    @pl.when(_NTRIP - 1 < nch)
    def _():
        wait_scatters(1)

    plsc.subcore_barrier()

    @pl.when(s < _NSTAGE)
    def _():
        def wb_body(j, carry):
            rr = r0 + j * _SC
            pltpu.sync_copy(acc.at[pl.ds(rr, _SC)], csA.at[pl.ds(0, _SC)])
            pltpu.sync_copy(csA.at[pl.ds(0, _SC)],
                            out_hbm.at[pl.ds(c * _N + rr, _SC)])
            return carry

        lax.fori_loop(0, _RPT // _SC, wb_body, 0)


_sheaf_step = functools.partial(
    pl.kernel,
    out_type=jax.ShapeDtypeStruct((_NC * _N, _BD), jnp.float32),
    mesh=plsc.VectorSubcoreMesh(core_axis_name="c", subcore_axis_name="s"),
    scratch_types=[
        pltpu.VMEM_SHARED((_N, _BD), jnp.float32),
        pltpu.VMEM((2 * _CB, _EBLK), jnp.int32),
        pltpu.VMEM((2 * _CB, _EBLK), jnp.int32),
        pltpu.VMEM((2 * _CB, _EBLK), jnp.int32),
        pltpu.VMEM((2 * _CB, _EBLK), jnp.int32),
        pltpu.VMEM((4, 2048), jnp.float32),
        pltpu.VMEM((4, 2048), jnp.float32),
        pltpu.VMEM((4, 2048), jnp.float32),
        pltpu.VMEM((4, 2048), jnp.float32),
        pltpu.VMEM((_C, _BD), jnp.float32),
        pltpu.VMEM((_C, _BD), jnp.float32),
        pltpu.VMEM((_C, _BD), jnp.float32),
        pltpu.VMEM((_C, _BD), jnp.float32),
        pltpu.VMEM((_C, _BD), jnp.float32),
        pltpu.VMEM((_C, _BD), jnp.float32),
        pltpu.VMEM((_C, _BD), jnp.float32),
        pltpu.VMEM((_C, _BD), jnp.float32),
        pltpu.SemaphoreType.DMA,
        pltpu.SemaphoreType.DMA,
        pltpu.SemaphoreType.DMA,
        pltpu.SemaphoreType.DMA,
        pltpu.SemaphoreType.DMA,
        pltpu.SemaphoreType.DMA,
        pltpu.SemaphoreType.DMA,
        pltpu.SemaphoreType.DMA,
    ],
    compiler_params=pltpu.CompilerParams(
        needs_layout_passes=False, use_tc_tiling_on_sc=False),
)(_sheaf_step_body)


def kernel(c0, src, dst, R_src, R_dst, poly_coeffs):
    B, N, D = c0.shape
    E = src.shape[0]
    p = jnp.transpose(c0, (1, 0, 2)).reshape(N, B * D)
    srcr = src.astype(jnp.int32).reshape(_NB, _EBLK)
    dstr = dst.astype(jnp.int32).reshape(_NB, _EBLK)
    rs = R_src.reshape(_NB, _EBLK, 4, 4).transpose(2, 0, 3, 1).reshape(-1)
    rd = R_dst.reshape(_NB, _EBLK, 4, 4).transpose(2, 0, 3, 1).reshape(-1)
    zero = jnp.zeros((N, B * D), jnp.float32)

    out = poly_coeffs[0] * p
    v = p
    for k in range(1, 4):
        parts = _sheaf_step(v, srcr, dstr, rs, rd, zero)
        v = parts[:N] + parts[N:]
        out = out + poly_coeffs[k] * v
    return out.reshape(N, B, D).transpose(1, 0, 2)
